# baseline (speedup 1.0000x reference)
import jax
import jax.numpy as jnp
from jax.experimental import pallas as pl
from jax.experimental.pallas import tpu as pltpu

_ROWS = 16384
_COLS = 2048
_BLOCK_ROWS = 512
_NB = _ROWS // _BLOCK_ROWS


def _quant_body(nl_ref, x_ref, o_ref, mm_ref):
    p = pl.program_id(0)
    i = pl.program_id(1)

    @pl.when(p == 0)
    def _reduce_phase():
        @pl.when(i == 0)
        def _init():
            mm_ref[0] = jnp.inf
            mm_ref[1] = -jnp.inf

        x = x_ref[...]
        mm_ref[0] = jnp.minimum(mm_ref[0], jnp.min(x))
        mm_ref[1] = jnp.maximum(mm_ref[1], jnp.max(x))

    @pl.when(p == 1)
    def _quantize_phase():
        nl = nl_ref[0]
        rng = mm_ref[1] - mm_ref[0]
        scale = rng / nl
        inv_scale = nl / rng
        o_ref[...] = jnp.round(x_ref[...] * inv_scale) * scale


def kernel(input, bits):
    nlevels = (jnp.exp2(bits.astype(jnp.float32)) - 1.0
               if hasattr(bits, "astype")
               else jnp.float32(2.0 ** bits - 1.0))
    nlevels = jnp.reshape(nlevels, (1,))
    x2 = input.reshape(_ROWS, _COLS)
    out = pl.pallas_call(
        _quant_body,
        grid=(2, _NB),
        in_specs=[
            pl.BlockSpec(memory_space=pltpu.SMEM),
            pl.BlockSpec((_BLOCK_ROWS, _COLS), lambda p, i: (i, 0)),
        ],
        out_specs=pl.BlockSpec((_BLOCK_ROWS, _COLS), lambda p, i: (p * i, 0)),
        out_shape=jax.ShapeDtypeStruct((_ROWS, _COLS), jnp.float32),
        scratch_shapes=[pltpu.SMEM((2,), jnp.float32)],
    )(nlevels, x2)
    return out.reshape(input.shape)


# speedup vs baseline: 1.0800x; 1.0800x over previous
import jax
import jax.numpy as jnp
from jax.experimental import pallas as pl
from jax.experimental.pallas import tpu as pltpu

_ROWS = 16384
_COLS = 2048
_BLOCK_ROWS = 1024
_NB = _ROWS // _BLOCK_ROWS


def _quant_body(nl_ref, x_ref, o_ref, mm_ref):
    p = pl.program_id(0)
    i = pl.program_id(1)

    @pl.when(p == 0)
    def _reduce_phase():
        @pl.when(i == 0)
        def _init():
            mm_ref[0] = jnp.inf
            mm_ref[1] = -jnp.inf

        x = x_ref[...]
        mm_ref[0] = jnp.minimum(mm_ref[0], jnp.min(x))
        mm_ref[1] = jnp.maximum(mm_ref[1], jnp.max(x))

    @pl.when(p == 1)
    def _quantize_phase():
        nl = nl_ref[0]
        rng = mm_ref[1] - mm_ref[0]
        scale = rng / nl
        inv_scale = nl / rng
        o_ref[...] = jnp.round(x_ref[...] * inv_scale) * scale


def kernel(input, bits):
    nlevels = (jnp.exp2(bits.astype(jnp.float32)) - 1.0
               if hasattr(bits, "astype")
               else jnp.float32(2.0 ** bits - 1.0))
    nlevels = jnp.reshape(nlevels, (1,))
    x2 = input.reshape(_ROWS, _COLS)
    out = pl.pallas_call(
        _quant_body,
        grid=(2, _NB),
        in_specs=[
            pl.BlockSpec(memory_space=pltpu.SMEM),
            pl.BlockSpec((_BLOCK_ROWS, _COLS), lambda p, i: (i, 0)),
        ],
        out_specs=pl.BlockSpec((_BLOCK_ROWS, _COLS), lambda p, i: (p * i, 0)),
        out_shape=jax.ShapeDtypeStruct((_ROWS, _COLS), jnp.float32),
        scratch_shapes=[pltpu.SMEM((2,), jnp.float32)],
    )(nlevels, x2)
    return out.reshape(input.shape)
